# baseline (speedup 1.0000x reference)
import functools

import jax
import jax.numpy as jnp
from jax import lax
from jax.experimental import pallas as pl
from jax.experimental.pallas import tpu as pltpu
from jax.experimental.pallas import tpu_sc as plsc

N = 10000
NPAD = 10240
E = 320000
NC = 2
NS = 16
NW = NC * NS
CH = 128
NCHUNK = 80
EPW = NCHUNK * CH
EPAD = NW * EPW
W128 = 128
ZR = 64
ROWS_PER = NPAD // NS

_MESH = dict(core_axis_name="c", subcore_axis_name="s", num_cores=NC,
             num_subcores=NS)


PAN = 8
ZRS = 32
NCHT = NW * NCHUNK
CNT_FAST = 152
CNT_SLOW = 8
FAST_C = 1
assert NS * (CNT_FAST + CNT_SLOW) == NCHT


def _spmm_pipeline(cnt, start, tab_hbm, src_g, dst_g, pan_src, pan_dst, rows,
                   acc_sh, gsem, ssem, psem):
  def pan_descs(q):
    pb = q & 1
    sl = pl.ds(start + q * PAN, PAN)
    return (pltpu.make_async_copy(src_g.at[sl], pan_src[pb], psem.at[pb]),
            pltpu.make_async_copy(dst_g.at[sl], pan_dst[pb], psem.at[pb]))

  def g_desc(j, b):
    pb, r = (j // PAN) & 1, j % PAN
    return pltpu.make_async_copy(tab_hbm.at[pan_src[pb].at[r]], rows[b],
                                 gsem.at[b])

  def s_desc(j, b):
    pb, r = (j // PAN) & 1, j % PAN
    return pltpu.make_async_copy(rows[b], acc_sh.at[pan_dst[pb].at[r]],
                                 ssem.at[b])

  for d in pan_descs(0):
    d.start()
  for d in pan_descs(0):
    d.wait()
  g_desc(0, 0).start()
  for j in range(cnt):
    b = j & 1
    if j + 1 < cnt:
      if j >= 1:
        s_desc(j - 1, 1 - b).wait()
      if j % PAN == 0 and j + PAN < cnt:
        for d in pan_descs(j // PAN + 1):
          d.start()
      if (j + 1) % PAN == 0:
        for d in pan_descs((j + 1) // PAN):
          d.wait()
      g_desc(j + 1, 1 - b).start()
    g_desc(j, b).wait()
    s_desc(j, b).start(add=True)
  s_desc(cnt - 2, (cnt - 2) & 1).wait()
  s_desc(cnt - 1, (cnt - 1) & 1).wait()


def _spmm_body(tab_hbm, src_g, dst_g, zeros_hbm, out_hbm, ps0, ps1, pd0,
               pd1, rows0, rows1, zb_v, acc_sh, gsem, ssem, psem):
  c = lax.axis_index("c")
  s = lax.axis_index("s")
  pan_src = (ps0, ps1)
  pan_dst = (pd0, pd1)
  rows = (rows0, rows1)

  pltpu.sync_copy(zeros_hbm, zb_v)
  base = s * ROWS_PER
  for k in range(ROWS_PER // ZRS):
    pltpu.sync_copy(zb_v, acc_sh.at[pl.ds(base + k * ZRS, ZRS)])
  plsc.subcore_barrier()

  args = (tab_hbm, src_g, dst_g, pan_src, pan_dst, rows, acc_sh, gsem,
          ssem, psem)

  @pl.when(c == FAST_C)
  def _():
    _spmm_pipeline(CNT_FAST, s * CNT_FAST, *args)

  if CNT_SLOW:
    @pl.when(c != FAST_C)
    def _():
      _spmm_pipeline(CNT_SLOW, NS * CNT_FAST + s * CNT_SLOW, *args)

  plsc.subcore_barrier()
  for k in range(ROWS_PER // ZRS):
    off = base + k * ZRS
    pltpu.sync_copy(acc_sh.at[pl.ds(off, ZRS)], zb_v)
    pltpu.sync_copy(zb_v, out_hbm.at[c, pl.ds(off, ZRS)])


_spmm = functools.partial(
    pl.kernel,
    out_type=jax.ShapeDtypeStruct((NC, NPAD, W128), jnp.float32),
    mesh=plsc.VectorSubcoreMesh(**_MESH),
    scratch_types=[
        pltpu.VMEM((PAN, CH), jnp.int32),
        pltpu.VMEM((PAN, CH), jnp.int32),
        pltpu.VMEM((PAN, CH), jnp.int32),
        pltpu.VMEM((PAN, CH), jnp.int32),
        pltpu.VMEM((CH, W128), jnp.float32),
        pltpu.VMEM((CH, W128), jnp.float32),
        pltpu.VMEM((ZRS, W128), jnp.float32),
        pltpu.VMEM_SHARED((NPAD, W128), jnp.float32),
        pltpu.SemaphoreType.DMA((2,)),
        pltpu.SemaphoreType.DMA((2,)),
        pltpu.SemaphoreType.DMA((2,)),
    ],
)(_spmm_body)


def _deg_body(dst_hbm, ones_hbm, zeros_hbm, out_hbm, idst_v, ones_v, zb_v,
              acc_sh):
  c = lax.axis_index("c")
  s = lax.axis_index("s")
  wid = s * NC + c
  pltpu.sync_copy(ones_hbm, ones_v)
  pltpu.sync_copy(zeros_hbm, zb_v)
  base = s * ROWS_PER
  for k in range(ROWS_PER // ZR):
    pltpu.sync_copy(zb_v, acc_sh.at[pl.ds(base + k * ZR, ZR)])
  plsc.subcore_barrier()

  def chunk(j, carry):
    pltpu.sync_copy(dst_hbm.at[wid, j], idst_v)
    pltpu.sync_copy(ones_v, acc_sh.at[idst_v], add=True)
    return carry

  lax.fori_loop(0, NCHUNK, chunk, 0)
  plsc.subcore_barrier()
  for k in range(ROWS_PER // ZR):
    off = base + k * ZR
    pltpu.sync_copy(acc_sh.at[pl.ds(off, ZR)], zb_v)
    pltpu.sync_copy(zb_v, out_hbm.at[c, pl.ds(off, ZR)])


_deg_kernel = functools.partial(
    pl.kernel,
    out_type=jax.ShapeDtypeStruct((NC, NPAD, W128), jnp.float32),
    mesh=plsc.VectorSubcoreMesh(**_MESH),
    scratch_types=[
        pltpu.VMEM((CH,), jnp.int32),
        pltpu.VMEM((CH, W128), jnp.float32),
        pltpu.VMEM((ZR, W128), jnp.float32),
        pltpu.VMEM_SHARED((NPAD, W128), jnp.float32),
    ],
)(_deg_body)


BLK = 512


def _tc1_body(x_ref, d0_ref, d1_ref, o_ref):
  deg = d0_ref[:, :1] + d1_ref[:, :1] + 1.0
  dinv = lax.rsqrt(deg)
  o_ref[...] = x_ref[...] * dinv


_tc1 = pl.pallas_call(
    _tc1_body,
    grid=(NPAD // BLK,),
    in_specs=[
        pl.BlockSpec((BLK, W128), lambda i: (i, 0)),
        pl.BlockSpec((BLK, W128), lambda i: (i, 0)),
        pl.BlockSpec((BLK, W128), lambda i: (i, 0)),
    ],
    out_specs=pl.BlockSpec((BLK, W128), lambda i: (i, 0)),
    out_shape=jax.ShapeDtypeStruct((NPAD, W128), jnp.float32),
)


def _tc3_body(a0_ref, a1_ref, gx_ref, d0_ref, d1_ref, w1_ref, b1_ref, w2_ref,
              o_ref):
  deg = d0_ref[:, :1] + d1_ref[:, :1] + 1.0
  dinv = lax.rsqrt(deg)
  s1 = a0_ref[...] + a1_ref[...] + gx_ref[...]
  h1 = jnp.dot(s1, w1_ref[...], preferred_element_type=jnp.float32)
  z1 = jnp.maximum(h1 * dinv + b1_ref[...], 0.0)
  h2 = jnp.dot(z1, w2_ref[...], preferred_element_type=jnp.float32)
  g2 = h2 * dinv
  o_ref[...] = jnp.concatenate(
      [g2, jnp.zeros((g2.shape[0], W128 - g2.shape[1]), jnp.float32)], axis=1)


_tc3 = pl.pallas_call(
    _tc3_body,
    grid=(NPAD // BLK,),
    in_specs=[
        pl.BlockSpec((BLK, W128), lambda i: (i, 0)),
        pl.BlockSpec((BLK, W128), lambda i: (i, 0)),
        pl.BlockSpec((BLK, W128), lambda i: (i, 0)),
        pl.BlockSpec((BLK, W128), lambda i: (i, 0)),
        pl.BlockSpec((BLK, W128), lambda i: (i, 0)),
        pl.BlockSpec((W128, 32), lambda i: (0, 0)),
        pl.BlockSpec((1, 32), lambda i: (0, 0)),
        pl.BlockSpec((32, 64), lambda i: (0, 0)),
    ],
    out_specs=pl.BlockSpec((BLK, W128), lambda i: (i, 0)),
    out_shape=jax.ShapeDtypeStruct((NPAD, W128), jnp.float32),
)


def _tc5_body(a0_ref, a1_ref, g2_ref, d0_ref, d1_ref, b2_ref, w3_ref,
              b3_ref, w4_ref, b4_ref, o_ref):
  deg = d0_ref[:, :1] + d1_ref[:, :1] + 1.0
  dinv = lax.rsqrt(deg)
  s2 = a0_ref[...] + a1_ref[...] + g2_ref[...]
  z2 = jnp.maximum(s2 * dinv + b2_ref[...], 0.0)
  h = jnp.maximum(
      jnp.dot(z2, w3_ref[...], preferred_element_type=jnp.float32)
      + b3_ref[...], 0.0)
  o = jnp.dot(h, w4_ref[...], preferred_element_type=jnp.float32) + b4_ref[...]
  rows = lax.broadcasted_iota(jnp.int32, (NPAD, 8), 0)
  o = jnp.where(rows < N, o, -jnp.inf)
  m = jnp.max(o)
  p = jnp.exp(o - m)
  o_ref[...] = p / jnp.sum(p)


_tc5 = pl.pallas_call(
    _tc5_body,
    out_shape=jax.ShapeDtypeStruct((NPAD, 8), jnp.float32),
)


def kernel(x, edge_index, num_nodes, W1, b1, W2, b2, W3, b3, W4, b4):
  src = edge_index[0]
  dst = edge_index[1]
  pad = jnp.full((EPAD - E,), N, jnp.int32)
  src_p = jnp.concatenate([src, pad]).reshape(NW, NCHUNK, CH)
  dst_p = jnp.concatenate([dst, pad]).reshape(NW, NCHUNK, CH)
  x_p = jnp.pad(x, ((0, NPAD - N), (0, 0)))
  ones_c = jnp.ones((CH, W128), jnp.float32)
  zeros_c = jnp.zeros((ZR, W128), jnp.float32)
  zeros_s = jnp.zeros((ZRS, W128), jnp.float32)

  src_g = src_p.reshape(NCHT, CH)
  dst_g = dst_p.reshape(NCHT, CH)

  degp = _deg_kernel(dst_p, ones_c, zeros_c)
  d0, d1 = degp[0], degp[1]
  gx = _tc1(x_p, d0, d1)
  acc1 = _spmm(gx, src_g, dst_g, zeros_s)
  b2p = jnp.concatenate([b2, jnp.zeros((W128 - 64,), jnp.float32)])
  w3p = jnp.pad(W3, ((0, W128 - 64), (0, 0)))
  g2 = _tc3(acc1[0], acc1[1], gx, d0, d1, W1, b1.reshape(1, -1), W2)
  acc2 = _spmm(g2, src_g, dst_g, zeros_s)
  probs = _tc5(acc2[0], acc2[1], g2, d0, d1, b2p.reshape(1, -1), w3p,
               b3.reshape(1, -1), W4, b4.reshape(1, -1))
  return probs[:N].reshape(1, N * 8)


# speedup vs baseline: 1.0347x; 1.0347x over previous
import functools

import jax
import jax.numpy as jnp
from jax import lax
from jax.experimental import pallas as pl
from jax.experimental.pallas import tpu as pltpu
from jax.experimental.pallas import tpu_sc as plsc

N = 10000
NPAD = 10240
E = 320000
NC = 2
NS = 16
NW = NC * NS
CH = 128
NCHUNK = 80
EPW = NCHUNK * CH
EPAD = NW * EPW
W128 = 128
ZR = 64
ROWS_PER = NPAD // NS

_MESH = dict(core_axis_name="c", subcore_axis_name="s", num_cores=NC,
             num_subcores=NS)


PAN = 8
ZRS = 32
NCHT = NW * NCHUNK
CNT_FAST = 152
CNT_SLOW = 8
FAST_C = 1
assert NS * (CNT_FAST + CNT_SLOW) == NCHT


def _spmm_pipeline(cnt, start, tab_hbm, src_g, dst_g, pan_src, pan_dst, rows,
                   acc_sh, gsem, ssem, psem):
  def pan_descs(q):
    pb = q & 1
    sl = pl.ds(start + q * PAN, PAN)
    return (pltpu.make_async_copy(src_g.at[sl], pan_src[pb], psem.at[pb]),
            pltpu.make_async_copy(dst_g.at[sl], pan_dst[pb], psem.at[pb]))

  def g_desc(j, b):
    pb, r = (j // PAN) & 1, j % PAN
    return pltpu.make_async_copy(tab_hbm.at[pan_src[pb].at[r]], rows[b],
                                 gsem.at[b])

  def s_desc(j, b):
    pb, r = (j // PAN) & 1, j % PAN
    return pltpu.make_async_copy(rows[b], acc_sh.at[pan_dst[pb].at[r]],
                                 ssem.at[b])

  for d in pan_descs(0):
    d.start()
  for d in pan_descs(0):
    d.wait()
  g_desc(0, 0).start()
  for j in range(cnt):
    b = j & 1
    if j + 1 < cnt:
      if j >= 1:
        s_desc(j - 1, 1 - b).wait()
      if j % PAN == 0 and j + PAN < cnt:
        for d in pan_descs(j // PAN + 1):
          d.start()
      if (j + 1) % PAN == 0:
        for d in pan_descs((j + 1) // PAN):
          d.wait()
      g_desc(j + 1, 1 - b).start()
    g_desc(j, b).wait()
    s_desc(j, b).start(add=True)
  s_desc(cnt - 2, (cnt - 2) & 1).wait()
  s_desc(cnt - 1, (cnt - 1) & 1).wait()


def _spmm_body(tab_hbm, src_g, dst_g, zeros_hbm, out_hbm, ps0, ps1, pd0,
               pd1, rows0, rows1, zb_v, acc_sh, gsem, ssem, psem):
  c = lax.axis_index("c")
  s = lax.axis_index("s")
  pan_src = (ps0, ps1)
  pan_dst = (pd0, pd1)
  rows = (rows0, rows1)

  pltpu.sync_copy(zeros_hbm, zb_v)
  base = s * ROWS_PER
  for k in range(ROWS_PER // ZRS):
    pltpu.sync_copy(zb_v, acc_sh.at[pl.ds(base + k * ZRS, ZRS)])
  plsc.subcore_barrier()

  args = (tab_hbm, src_g, dst_g, pan_src, pan_dst, rows, acc_sh, gsem,
          ssem, psem)

  @pl.when(c == FAST_C)
  def _():
    _spmm_pipeline(CNT_FAST, s * CNT_FAST, *args)

  if CNT_SLOW:
    @pl.when(c != FAST_C)
    def _():
      _spmm_pipeline(CNT_SLOW, NS * CNT_FAST + s * CNT_SLOW, *args)

  plsc.subcore_barrier()
  for k in range(ROWS_PER // ZRS):
    off = base + k * ZRS
    pltpu.sync_copy(acc_sh.at[pl.ds(off, ZRS)], zb_v)
    pltpu.sync_copy(zb_v, out_hbm.at[c, pl.ds(off, ZRS)])


_spmm = functools.partial(
    pl.kernel,
    out_type=jax.ShapeDtypeStruct((NC, NPAD, W128), jnp.float32),
    mesh=plsc.VectorSubcoreMesh(**_MESH),
    scratch_types=[
        pltpu.VMEM((PAN, CH), jnp.int32),
        pltpu.VMEM((PAN, CH), jnp.int32),
        pltpu.VMEM((PAN, CH), jnp.int32),
        pltpu.VMEM((PAN, CH), jnp.int32),
        pltpu.VMEM((CH, W128), jnp.float32),
        pltpu.VMEM((CH, W128), jnp.float32),
        pltpu.VMEM((ZRS, W128), jnp.float32),
        pltpu.VMEM_SHARED((NPAD, W128), jnp.float32),
        pltpu.SemaphoreType.DMA((2,)),
        pltpu.SemaphoreType.DMA((2,)),
        pltpu.SemaphoreType.DMA((2,)),
    ],
)(_spmm_body)


def _deg_body(dst_hbm, ones_hbm, zeros_hbm, out_hbm, idst0, idst1, ones_v,
              zb_v, acc_sh, isem, ssem):
  c = lax.axis_index("c")
  s = lax.axis_index("s")
  wid = s * NC + c
  idst = (idst0, idst1)

  def i_desc(j, b):
    return pltpu.make_async_copy(dst_hbm.at[wid, j], idst[b], isem.at[b])

  def s_desc(j, b):
    return pltpu.make_async_copy(ones_v, acc_sh.at[idst[b]], ssem.at[b])

  i_desc(0, 0).start()
  pltpu.sync_copy(ones_hbm, ones_v)
  pltpu.sync_copy(zeros_hbm, zb_v)
  base = s * ROWS_PER
  for k in range(ROWS_PER // ZR):
    pltpu.sync_copy(zb_v, acc_sh.at[pl.ds(base + k * ZR, ZR)])
  plsc.subcore_barrier()

  for j in range(NCHUNK):
    b = j & 1
    if j + 1 < NCHUNK:
      if j >= 1:
        s_desc(j - 1, 1 - b).wait()
      i_desc(j + 1, 1 - b).start()
    i_desc(j, b).wait()
    s_desc(j, b).start(add=True)
  s_desc(NCHUNK - 2, (NCHUNK - 2) & 1).wait()
  s_desc(NCHUNK - 1, (NCHUNK - 1) & 1).wait()
  plsc.subcore_barrier()
  for k in range(ROWS_PER // ZR):
    off = base + k * ZR
    pltpu.sync_copy(acc_sh.at[pl.ds(off, ZR)], zb_v)
    pltpu.sync_copy(zb_v, out_hbm.at[c, pl.ds(off, ZR)])


_deg_kernel = functools.partial(
    pl.kernel,
    out_type=jax.ShapeDtypeStruct((NC, NPAD, W128), jnp.float32),
    mesh=plsc.VectorSubcoreMesh(**_MESH),
    scratch_types=[
        pltpu.VMEM((CH,), jnp.int32),
        pltpu.VMEM((CH,), jnp.int32),
        pltpu.VMEM((CH, W128), jnp.float32),
        pltpu.VMEM((ZR, W128), jnp.float32),
        pltpu.VMEM_SHARED((NPAD, W128), jnp.float32),
        pltpu.SemaphoreType.DMA((2,)),
        pltpu.SemaphoreType.DMA((2,)),
    ],
)(_deg_body)


BLK = 512


def _tc1_body(x_ref, d0_ref, d1_ref, o_ref):
  deg = d0_ref[:, :1] + d1_ref[:, :1] + 1.0
  dinv = lax.rsqrt(deg)
  o_ref[...] = x_ref[...] * dinv


_tc1 = pl.pallas_call(
    _tc1_body,
    grid=(NPAD // BLK,),
    in_specs=[
        pl.BlockSpec((BLK, W128), lambda i: (i, 0)),
        pl.BlockSpec((BLK, W128), lambda i: (i, 0)),
        pl.BlockSpec((BLK, W128), lambda i: (i, 0)),
    ],
    out_specs=pl.BlockSpec((BLK, W128), lambda i: (i, 0)),
    out_shape=jax.ShapeDtypeStruct((NPAD, W128), jnp.float32),
)


def _tc3_body(a0_ref, a1_ref, gx_ref, d0_ref, d1_ref, w1_ref, b1_ref, w2_ref,
              o_ref):
  deg = d0_ref[:, :1] + d1_ref[:, :1] + 1.0
  dinv = lax.rsqrt(deg)
  s1 = a0_ref[...] + a1_ref[...] + gx_ref[...]
  h1 = jnp.dot(s1, w1_ref[...], preferred_element_type=jnp.float32)
  z1 = jnp.maximum(h1 * dinv + b1_ref[...], 0.0)
  h2 = jnp.dot(z1, w2_ref[...], preferred_element_type=jnp.float32)
  g2 = h2 * dinv
  o_ref[...] = jnp.concatenate(
      [g2, jnp.zeros((g2.shape[0], W128 - g2.shape[1]), jnp.float32)], axis=1)


_tc3 = pl.pallas_call(
    _tc3_body,
    grid=(NPAD // BLK,),
    in_specs=[
        pl.BlockSpec((BLK, W128), lambda i: (i, 0)),
        pl.BlockSpec((BLK, W128), lambda i: (i, 0)),
        pl.BlockSpec((BLK, W128), lambda i: (i, 0)),
        pl.BlockSpec((BLK, W128), lambda i: (i, 0)),
        pl.BlockSpec((BLK, W128), lambda i: (i, 0)),
        pl.BlockSpec((W128, 32), lambda i: (0, 0)),
        pl.BlockSpec((1, 32), lambda i: (0, 0)),
        pl.BlockSpec((32, 64), lambda i: (0, 0)),
    ],
    out_specs=pl.BlockSpec((BLK, W128), lambda i: (i, 0)),
    out_shape=jax.ShapeDtypeStruct((NPAD, W128), jnp.float32),
)


def _tc5_body(a0_ref, a1_ref, g2_ref, d0_ref, d1_ref, b2_ref, w3_ref,
              b3_ref, w4_ref, b4_ref, o_ref):
  deg = d0_ref[:, :1] + d1_ref[:, :1] + 1.0
  dinv = lax.rsqrt(deg)
  s2 = a0_ref[...] + a1_ref[...] + g2_ref[...]
  z2 = jnp.maximum(s2 * dinv + b2_ref[...], 0.0)
  h = jnp.maximum(
      jnp.dot(z2, w3_ref[...], preferred_element_type=jnp.float32)
      + b3_ref[...], 0.0)
  o = jnp.dot(h, w4_ref[...], preferred_element_type=jnp.float32) + b4_ref[...]
  rows = lax.broadcasted_iota(jnp.int32, (NPAD, 8), 0)
  o = jnp.where(rows < N, o, -jnp.inf)
  m = jnp.max(o)
  p = jnp.exp(o - m)
  o_ref[...] = p / jnp.sum(p)


_tc5 = pl.pallas_call(
    _tc5_body,
    out_shape=jax.ShapeDtypeStruct((NPAD, 8), jnp.float32),
)


def kernel(x, edge_index, num_nodes, W1, b1, W2, b2, W3, b3, W4, b4):
  src = edge_index[0]
  dst = edge_index[1]
  pad = jnp.full((EPAD - E,), N, jnp.int32)
  src_p = jnp.concatenate([src, pad]).reshape(NW, NCHUNK, CH)
  dst_p = jnp.concatenate([dst, pad]).reshape(NW, NCHUNK, CH)
  x_p = jnp.pad(x, ((0, NPAD - N), (0, 0)))
  ones_c = jnp.ones((CH, W128), jnp.float32)
  zeros_c = jnp.zeros((ZR, W128), jnp.float32)
  zeros_s = jnp.zeros((ZRS, W128), jnp.float32)

  src_g = src_p.reshape(NCHT, CH)
  dst_g = dst_p.reshape(NCHT, CH)

  degp = _deg_kernel(dst_p, ones_c, zeros_c)
  d0, d1 = degp[0], degp[1]
  gx = _tc1(x_p, d0, d1)
  acc1 = _spmm(gx, src_g, dst_g, zeros_s)
  b2p = jnp.concatenate([b2, jnp.zeros((W128 - 64,), jnp.float32)])
  w3p = jnp.pad(W3, ((0, W128 - 64), (0, 0)))
  g2 = _tc3(acc1[0], acc1[1], gx, d0, d1, W1, b1.reshape(1, -1), W2)
  acc2 = _spmm(g2, src_g, dst_g, zeros_s)
  probs = _tc5(acc2[0], acc2[1], g2, d0, d1, b2p.reshape(1, -1), w3p,
               b3.reshape(1, -1), W4, b4.reshape(1, -1))
  return probs[:N].reshape(1, N * 8)


# speedup vs baseline: 1.0502x; 1.0150x over previous
import functools

import jax
import jax.numpy as jnp
from jax import lax
from jax.experimental import pallas as pl
from jax.experimental.pallas import tpu as pltpu
from jax.experimental.pallas import tpu_sc as plsc

N = 10000
NPAD = 10240
E = 320000
NC = 2
NS = 16
NW = NC * NS
CH = 128
NCHUNK = 80
EPW = NCHUNK * CH
EPAD = NW * EPW
W128 = 128
ZR = 64
ROWS_PER = NPAD // NS

_MESH = dict(core_axis_name="c", subcore_axis_name="s", num_cores=NC,
             num_subcores=NS)


PAN = 8
ZRS = 32
NCHT = NW * NCHUNK
CNT_FAST = 152
CNT_SLOW = 8
FAST_C = 1
assert NS * (CNT_FAST + CNT_SLOW) == NCHT


WB = 128


def _writeout(acc_sh, out_hbm, c, base, bufs, sem_a, sem_b):
  ep = ROWS_PER // WB

  def r_desc(k, b):
    return pltpu.make_async_copy(acc_sh.at[pl.ds(base + k * WB, WB)],
                                 bufs[b], sem_a.at[b])

  def w_desc(k, b):
    return pltpu.make_async_copy(bufs[b],
                                 out_hbm.at[c, pl.ds(base + k * WB, WB)],
                                 sem_b.at[b])

  r_desc(0, 0).start()
  for k in range(ep):
    b = k & 1
    if k + 1 < ep:
      if k >= 1:
        w_desc(k - 1, 1 - b).wait()
      r_desc(k + 1, 1 - b).start()
    r_desc(k, b).wait()
    w_desc(k, b).start()
  w_desc(ep - 2, (ep - 2) & 1).wait()
  w_desc(ep - 1, (ep - 1) & 1).wait()


def _spmm_pipeline(cnt, start, tab_hbm, src_g, dst_g, pan_src, pan_dst, rows,
                   acc_sh, gsem, ssem, psem):
  def pan_descs(q):
    pb = q & 1
    sl = pl.ds(start + q * PAN, PAN)
    return (pltpu.make_async_copy(src_g.at[sl], pan_src[pb], psem.at[pb]),
            pltpu.make_async_copy(dst_g.at[sl], pan_dst[pb], psem.at[pb]))

  def g_desc(j, b):
    pb, r = (j // PAN) & 1, j % PAN
    return pltpu.make_async_copy(tab_hbm.at[pan_src[pb].at[r]], rows[b],
                                 gsem.at[b])

  def s_desc(j, b):
    pb, r = (j // PAN) & 1, j % PAN
    return pltpu.make_async_copy(rows[b], acc_sh.at[pan_dst[pb].at[r]],
                                 ssem.at[b])

  for d in pan_descs(0):
    d.start()
  for d in pan_descs(0):
    d.wait()
  g_desc(0, 0).start()
  for j in range(cnt):
    b = j & 1
    if j + 1 < cnt:
      if j >= 1:
        s_desc(j - 1, 1 - b).wait()
      if j % PAN == 0 and j + PAN < cnt:
        for d in pan_descs(j // PAN + 1):
          d.start()
      if (j + 1) % PAN == 0:
        for d in pan_descs((j + 1) // PAN):
          d.wait()
      g_desc(j + 1, 1 - b).start()
    g_desc(j, b).wait()
    s_desc(j, b).start(add=True)
  s_desc(cnt - 2, (cnt - 2) & 1).wait()
  s_desc(cnt - 1, (cnt - 1) & 1).wait()


def _spmm_body(tab_hbm, src_g, dst_g, zeros_hbm, out_hbm, ps0, ps1, pd0,
               pd1, rows0, rows1, zb_v, acc_sh, gsem, ssem, psem):
  c = lax.axis_index("c")
  s = lax.axis_index("s")
  pan_src = (ps0, ps1)
  pan_dst = (pd0, pd1)
  rows = (rows0, rows1)

  pltpu.sync_copy(zeros_hbm, zb_v)
  base = s * ROWS_PER
  for k in range(ROWS_PER // ZRS):
    pltpu.sync_copy(zb_v, acc_sh.at[pl.ds(base + k * ZRS, ZRS)])
  plsc.subcore_barrier()

  args = (tab_hbm, src_g, dst_g, pan_src, pan_dst, rows, acc_sh, gsem,
          ssem, psem)

  @pl.when(c == FAST_C)
  def _():
    _spmm_pipeline(CNT_FAST, s * CNT_FAST, *args)

  if CNT_SLOW:
    @pl.when(c != FAST_C)
    def _():
      _spmm_pipeline(CNT_SLOW, NS * CNT_FAST + s * CNT_SLOW, *args)

  plsc.subcore_barrier()
  _writeout(acc_sh, out_hbm, c, base, rows, gsem, ssem)


_spmm = functools.partial(
    pl.kernel,
    out_type=jax.ShapeDtypeStruct((NC, NPAD, W128), jnp.float32),
    mesh=plsc.VectorSubcoreMesh(**_MESH),
    scratch_types=[
        pltpu.VMEM((PAN, CH), jnp.int32),
        pltpu.VMEM((PAN, CH), jnp.int32),
        pltpu.VMEM((PAN, CH), jnp.int32),
        pltpu.VMEM((PAN, CH), jnp.int32),
        pltpu.VMEM((CH, W128), jnp.float32),
        pltpu.VMEM((CH, W128), jnp.float32),
        pltpu.VMEM((ZRS, W128), jnp.float32),
        pltpu.VMEM_SHARED((NPAD, W128), jnp.float32),
        pltpu.SemaphoreType.DMA((2,)),
        pltpu.SemaphoreType.DMA((2,)),
        pltpu.SemaphoreType.DMA((2,)),
    ],
)(_spmm_body)


def _deg_body(dst_hbm, ones_hbm, zeros_hbm, out_hbm, idst0, idst1, ones_v,
              zb_v, zb2_v, acc_sh, isem, ssem):
  c = lax.axis_index("c")
  s = lax.axis_index("s")
  wid = s * NC + c
  idst = (idst0, idst1)

  def i_desc(j, b):
    return pltpu.make_async_copy(dst_hbm.at[wid, j], idst[b], isem.at[b])

  def s_desc(j, b):
    return pltpu.make_async_copy(ones_v, acc_sh.at[idst[b]], ssem.at[b])

  i_desc(0, 0).start()
  pltpu.sync_copy(ones_hbm, ones_v)
  pltpu.sync_copy(zeros_hbm, zb_v)
  base = s * ROWS_PER
  for k in range(ROWS_PER // ZR):
    pltpu.sync_copy(zb_v, acc_sh.at[pl.ds(base + k * ZR, ZR)])
  plsc.subcore_barrier()

  for j in range(NCHUNK):
    b = j & 1
    if j + 1 < NCHUNK:
      if j >= 1:
        s_desc(j - 1, 1 - b).wait()
      i_desc(j + 1, 1 - b).start()
    i_desc(j, b).wait()
    s_desc(j, b).start(add=True)
  s_desc(NCHUNK - 2, (NCHUNK - 2) & 1).wait()
  s_desc(NCHUNK - 1, (NCHUNK - 1) & 1).wait()
  plsc.subcore_barrier()
  _writeout(acc_sh, out_hbm, c, base, (ones_v, zb2_v), isem, ssem)


_deg_kernel = functools.partial(
    pl.kernel,
    out_type=jax.ShapeDtypeStruct((NC, NPAD, W128), jnp.float32),
    mesh=plsc.VectorSubcoreMesh(**_MESH),
    scratch_types=[
        pltpu.VMEM((CH,), jnp.int32),
        pltpu.VMEM((CH,), jnp.int32),
        pltpu.VMEM((CH, W128), jnp.float32),
        pltpu.VMEM((ZR, W128), jnp.float32),
        pltpu.VMEM((WB, W128), jnp.float32),
        pltpu.VMEM_SHARED((NPAD, W128), jnp.float32),
        pltpu.SemaphoreType.DMA((2,)),
        pltpu.SemaphoreType.DMA((2,)),
    ],
)(_deg_body)


BLK = 512


def _tc1_body(x_ref, d0_ref, d1_ref, o_ref):
  deg = d0_ref[:, :1] + d1_ref[:, :1] + 1.0
  dinv = lax.rsqrt(deg)
  o_ref[...] = x_ref[...] * dinv


_tc1 = pl.pallas_call(
    _tc1_body,
    grid=(NPAD // BLK,),
    in_specs=[
        pl.BlockSpec((BLK, W128), lambda i: (i, 0)),
        pl.BlockSpec((BLK, W128), lambda i: (i, 0)),
        pl.BlockSpec((BLK, W128), lambda i: (i, 0)),
    ],
    out_specs=pl.BlockSpec((BLK, W128), lambda i: (i, 0)),
    out_shape=jax.ShapeDtypeStruct((NPAD, W128), jnp.float32),
)


def _tc3_body(a0_ref, a1_ref, gx_ref, d0_ref, d1_ref, w1_ref, b1_ref, w2_ref,
              o_ref):
  deg = d0_ref[:, :1] + d1_ref[:, :1] + 1.0
  dinv = lax.rsqrt(deg)
  s1 = a0_ref[...] + a1_ref[...] + gx_ref[...]
  h1 = jnp.dot(s1, w1_ref[...], preferred_element_type=jnp.float32)
  z1 = jnp.maximum(h1 * dinv + b1_ref[...], 0.0)
  h2 = jnp.dot(z1, w2_ref[...], preferred_element_type=jnp.float32)
  g2 = h2 * dinv
  o_ref[...] = jnp.concatenate(
      [g2, jnp.zeros((g2.shape[0], W128 - g2.shape[1]), jnp.float32)], axis=1)


_tc3 = pl.pallas_call(
    _tc3_body,
    grid=(NPAD // BLK,),
    in_specs=[
        pl.BlockSpec((BLK, W128), lambda i: (i, 0)),
        pl.BlockSpec((BLK, W128), lambda i: (i, 0)),
        pl.BlockSpec((BLK, W128), lambda i: (i, 0)),
        pl.BlockSpec((BLK, W128), lambda i: (i, 0)),
        pl.BlockSpec((BLK, W128), lambda i: (i, 0)),
        pl.BlockSpec((W128, 32), lambda i: (0, 0)),
        pl.BlockSpec((1, 32), lambda i: (0, 0)),
        pl.BlockSpec((32, 64), lambda i: (0, 0)),
    ],
    out_specs=pl.BlockSpec((BLK, W128), lambda i: (i, 0)),
    out_shape=jax.ShapeDtypeStruct((NPAD, W128), jnp.float32),
)


def _tc5_body(a0_ref, a1_ref, g2_ref, d0_ref, d1_ref, b2_ref, w3_ref,
              b3_ref, w4_ref, b4_ref, o_ref):
  deg = d0_ref[:, :1] + d1_ref[:, :1] + 1.0
  dinv = lax.rsqrt(deg)
  s2 = a0_ref[...] + a1_ref[...] + g2_ref[...]
  z2 = jnp.maximum(s2 * dinv + b2_ref[...], 0.0)
  h = jnp.maximum(
      jnp.dot(z2, w3_ref[...], preferred_element_type=jnp.float32)
      + b3_ref[...], 0.0)
  o = jnp.dot(h, w4_ref[...], preferred_element_type=jnp.float32) + b4_ref[...]
  rows = lax.broadcasted_iota(jnp.int32, (NPAD, 8), 0)
  o = jnp.where(rows < N, o, -jnp.inf)
  m = jnp.max(o)
  p = jnp.exp(o - m)
  o_ref[...] = p / jnp.sum(p)


_tc5 = pl.pallas_call(
    _tc5_body,
    out_shape=jax.ShapeDtypeStruct((NPAD, 8), jnp.float32),
)


def kernel(x, edge_index, num_nodes, W1, b1, W2, b2, W3, b3, W4, b4):
  src = edge_index[0]
  dst = edge_index[1]
  pad = jnp.full((EPAD - E,), N, jnp.int32)
  src_p = jnp.concatenate([src, pad]).reshape(NW, NCHUNK, CH)
  dst_p = jnp.concatenate([dst, pad]).reshape(NW, NCHUNK, CH)
  x_p = jnp.pad(x, ((0, NPAD - N), (0, 0)))
  ones_c = jnp.ones((CH, W128), jnp.float32)
  zeros_c = jnp.zeros((ZR, W128), jnp.float32)
  zeros_s = jnp.zeros((ZRS, W128), jnp.float32)

  src_g = src_p.reshape(NCHT, CH)
  dst_g = dst_p.reshape(NCHT, CH)

  degp = _deg_kernel(dst_p, ones_c, zeros_c)
  d0, d1 = degp[0], degp[1]
  gx = _tc1(x_p, d0, d1)
  acc1 = _spmm(gx, src_g, dst_g, zeros_s)
  b2p = jnp.concatenate([b2, jnp.zeros((W128 - 64,), jnp.float32)])
  w3p = jnp.pad(W3, ((0, W128 - 64), (0, 0)))
  g2 = _tc3(acc1[0], acc1[1], gx, d0, d1, W1, b1.reshape(1, -1), W2)
  acc2 = _spmm(g2, src_g, dst_g, zeros_s)
  probs = _tc5(acc2[0], acc2[1], g2, d0, d1, b2p.reshape(1, -1), w3p,
               b3.reshape(1, -1), W4, b4.reshape(1, -1))
  return probs[:N].reshape(1, N * 8)


# speedup vs baseline: 1.0538x; 1.0035x over previous
import functools

import jax
import jax.numpy as jnp
from jax import lax
from jax.experimental import pallas as pl
from jax.experimental.pallas import tpu as pltpu
from jax.experimental.pallas import tpu_sc as plsc

N = 10000
NPAD = 10240
E = 320000
NC = 2
NS = 16
NW = NC * NS
CH = 128
NCHUNK = 80
EPW = NCHUNK * CH
EPAD = NW * EPW
W128 = 128
ZR = 64
ROWS_PER = NPAD // NS

_MESH = dict(core_axis_name="c", subcore_axis_name="s", num_cores=NC,
             num_subcores=NS)


PAN = 8
ZRS = 32
NCHT = NW * NCHUNK
CNT_FAST = 152
CNT_SLOW = 8
FAST_C = 1
assert NS * (CNT_FAST + CNT_SLOW) == NCHT


WB = 128


def _writeout(acc_sh, out_hbm, c, base, bufs, sem_a, sem_b):
  ep = ROWS_PER // WB

  def r_desc(k, b):
    return pltpu.make_async_copy(acc_sh.at[pl.ds(base + k * WB, WB)],
                                 bufs[b], sem_a.at[b])

  def w_desc(k, b):
    return pltpu.make_async_copy(bufs[b],
                                 out_hbm.at[c, pl.ds(base + k * WB, WB)],
                                 sem_b.at[b])

  r_desc(0, 0).start()
  for k in range(ep):
    b = k & 1
    if k + 1 < ep:
      if k >= 1:
        w_desc(k - 1, 1 - b).wait()
      r_desc(k + 1, 1 - b).start()
    r_desc(k, b).wait()
    w_desc(k, b).start()
  w_desc(ep - 2, (ep - 2) & 1).wait()
  w_desc(ep - 1, (ep - 1) & 1).wait()


def _zero_acc(acc_sh, base, zb_v, sem, zr):
  descs = [
      pltpu.make_async_copy(zb_v, acc_sh.at[pl.ds(base + k * zr, zr)],
                            sem.at[k & 1])
      for k in range(ROWS_PER // zr)
  ]
  for d in descs:
    d.start()
  for d in descs:
    d.wait()


def _spmm_pipeline(cnt, start, tab_hbm, src_g, dst_g, pan_src, pan_dst, rows,
                   acc_sh, gsem, ssem, psem):
  def pan_descs(q):
    pb = q & 1
    sl = pl.ds(start + q * PAN, PAN)
    return (pltpu.make_async_copy(src_g.at[sl], pan_src[pb], psem.at[pb]),
            pltpu.make_async_copy(dst_g.at[sl], pan_dst[pb], psem.at[pb]))

  def g_desc(j, b):
    pb, r = (j // PAN) & 1, j % PAN
    return pltpu.make_async_copy(tab_hbm.at[pan_src[pb].at[r]], rows[b],
                                 gsem.at[b])

  def s_desc(j, b):
    pb, r = (j // PAN) & 1, j % PAN
    return pltpu.make_async_copy(rows[b], acc_sh.at[pan_dst[pb].at[r]],
                                 ssem.at[b])

  for d in pan_descs(0):
    d.start()
  for d in pan_descs(0):
    d.wait()
  g_desc(0, 0).start()
  for j in range(cnt):
    b = j & 1
    if j + 1 < cnt:
      if j >= 1:
        s_desc(j - 1, 1 - b).wait()
      if j % PAN == 0 and j + PAN < cnt:
        for d in pan_descs(j // PAN + 1):
          d.start()
      if (j + 1) % PAN == 0:
        for d in pan_descs((j + 1) // PAN):
          d.wait()
      g_desc(j + 1, 1 - b).start()
    g_desc(j, b).wait()
    s_desc(j, b).start(add=True)
  s_desc(cnt - 2, (cnt - 2) & 1).wait()
  s_desc(cnt - 1, (cnt - 1) & 1).wait()


def _spmm_body(tab_hbm, src_g, dst_g, zeros_hbm, out_hbm, ps0, ps1, pd0,
               pd1, rows0, rows1, zb_v, acc_sh, gsem, ssem, psem):
  c = lax.axis_index("c")
  s = lax.axis_index("s")
  pan_src = (ps0, ps1)
  pan_dst = (pd0, pd1)
  rows = (rows0, rows1)

  pltpu.sync_copy(zeros_hbm, zb_v)
  base = s * ROWS_PER
  _zero_acc(acc_sh, base, zb_v, ssem, ZRS)
  plsc.subcore_barrier()

  args = (tab_hbm, src_g, dst_g, pan_src, pan_dst, rows, acc_sh, gsem,
          ssem, psem)

  @pl.when(c == FAST_C)
  def _():
    _spmm_pipeline(CNT_FAST, s * CNT_FAST, *args)

  if CNT_SLOW:
    @pl.when(c != FAST_C)
    def _():
      _spmm_pipeline(CNT_SLOW, NS * CNT_FAST + s * CNT_SLOW, *args)

  plsc.subcore_barrier()
  _writeout(acc_sh, out_hbm, c, base, rows, gsem, ssem)


_spmm = functools.partial(
    pl.kernel,
    out_type=jax.ShapeDtypeStruct((NC, NPAD, W128), jnp.float32),
    mesh=plsc.VectorSubcoreMesh(**_MESH),
    scratch_types=[
        pltpu.VMEM((PAN, CH), jnp.int32),
        pltpu.VMEM((PAN, CH), jnp.int32),
        pltpu.VMEM((PAN, CH), jnp.int32),
        pltpu.VMEM((PAN, CH), jnp.int32),
        pltpu.VMEM((CH, W128), jnp.float32),
        pltpu.VMEM((CH, W128), jnp.float32),
        pltpu.VMEM((ZRS, W128), jnp.float32),
        pltpu.VMEM_SHARED((NPAD, W128), jnp.float32),
        pltpu.SemaphoreType.DMA((2,)),
        pltpu.SemaphoreType.DMA((2,)),
        pltpu.SemaphoreType.DMA((2,)),
    ],
)(_spmm_body)


def _deg_body(dst_hbm, ones_hbm, zeros_hbm, out_hbm, idst0, idst1, ones_v,
              zb_v, zb2_v, acc_sh, isem, ssem):
  c = lax.axis_index("c")
  s = lax.axis_index("s")
  wid = s * NC + c
  idst = (idst0, idst1)

  def i_desc(j, b):
    return pltpu.make_async_copy(dst_hbm.at[wid, j], idst[b], isem.at[b])

  def s_desc(j, b):
    return pltpu.make_async_copy(ones_v, acc_sh.at[idst[b]], ssem.at[b])

  i_desc(0, 0).start()
  pltpu.sync_copy(ones_hbm, ones_v)
  pltpu.sync_copy(zeros_hbm, zb_v)
  base = s * ROWS_PER
  _zero_acc(acc_sh, base, zb_v, ssem, ZR)
  plsc.subcore_barrier()

  for j in range(NCHUNK):
    b = j & 1
    if j + 1 < NCHUNK:
      if j >= 1:
        s_desc(j - 1, 1 - b).wait()
      i_desc(j + 1, 1 - b).start()
    i_desc(j, b).wait()
    s_desc(j, b).start(add=True)
  s_desc(NCHUNK - 2, (NCHUNK - 2) & 1).wait()
  s_desc(NCHUNK - 1, (NCHUNK - 1) & 1).wait()
  plsc.subcore_barrier()
  _writeout(acc_sh, out_hbm, c, base, (ones_v, zb2_v), isem, ssem)


_deg_kernel = functools.partial(
    pl.kernel,
    out_type=jax.ShapeDtypeStruct((NC, NPAD, W128), jnp.float32),
    mesh=plsc.VectorSubcoreMesh(**_MESH),
    scratch_types=[
        pltpu.VMEM((CH,), jnp.int32),
        pltpu.VMEM((CH,), jnp.int32),
        pltpu.VMEM((CH, W128), jnp.float32),
        pltpu.VMEM((ZR, W128), jnp.float32),
        pltpu.VMEM((WB, W128), jnp.float32),
        pltpu.VMEM_SHARED((NPAD, W128), jnp.float32),
        pltpu.SemaphoreType.DMA((2,)),
        pltpu.SemaphoreType.DMA((2,)),
    ],
)(_deg_body)


BLK = 512


def _tc1_body(x_ref, d0_ref, d1_ref, o_ref):
  deg = d0_ref[:, :1] + d1_ref[:, :1] + 1.0
  dinv = lax.rsqrt(deg)
  o_ref[...] = x_ref[...] * dinv


_tc1 = pl.pallas_call(
    _tc1_body,
    grid=(NPAD // BLK,),
    in_specs=[
        pl.BlockSpec((BLK, W128), lambda i: (i, 0)),
        pl.BlockSpec((BLK, W128), lambda i: (i, 0)),
        pl.BlockSpec((BLK, W128), lambda i: (i, 0)),
    ],
    out_specs=pl.BlockSpec((BLK, W128), lambda i: (i, 0)),
    out_shape=jax.ShapeDtypeStruct((NPAD, W128), jnp.float32),
)


def _tc3_body(a0_ref, a1_ref, gx_ref, d0_ref, d1_ref, w1_ref, b1_ref, w2_ref,
              o_ref):
  deg = d0_ref[:, :1] + d1_ref[:, :1] + 1.0
  dinv = lax.rsqrt(deg)
  s1 = a0_ref[...] + a1_ref[...] + gx_ref[...]
  h1 = jnp.dot(s1, w1_ref[...], preferred_element_type=jnp.float32)
  z1 = jnp.maximum(h1 * dinv + b1_ref[...], 0.0)
  h2 = jnp.dot(z1, w2_ref[...], preferred_element_type=jnp.float32)
  g2 = h2 * dinv
  o_ref[...] = jnp.concatenate(
      [g2, jnp.zeros((g2.shape[0], W128 - g2.shape[1]), jnp.float32)], axis=1)


_tc3 = pl.pallas_call(
    _tc3_body,
    grid=(NPAD // BLK,),
    in_specs=[
        pl.BlockSpec((BLK, W128), lambda i: (i, 0)),
        pl.BlockSpec((BLK, W128), lambda i: (i, 0)),
        pl.BlockSpec((BLK, W128), lambda i: (i, 0)),
        pl.BlockSpec((BLK, W128), lambda i: (i, 0)),
        pl.BlockSpec((BLK, W128), lambda i: (i, 0)),
        pl.BlockSpec((W128, 32), lambda i: (0, 0)),
        pl.BlockSpec((1, 32), lambda i: (0, 0)),
        pl.BlockSpec((32, 64), lambda i: (0, 0)),
    ],
    out_specs=pl.BlockSpec((BLK, W128), lambda i: (i, 0)),
    out_shape=jax.ShapeDtypeStruct((NPAD, W128), jnp.float32),
)


def _tc5_body(a0_ref, a1_ref, g2_ref, d0_ref, d1_ref, b2_ref, w3_ref,
              b3_ref, w4_ref, b4_ref, o_ref):
  deg = d0_ref[:, :1] + d1_ref[:, :1] + 1.0
  dinv = lax.rsqrt(deg)
  s2 = a0_ref[...] + a1_ref[...] + g2_ref[...]
  z2 = jnp.maximum(s2 * dinv + b2_ref[...], 0.0)
  h = jnp.maximum(
      jnp.dot(z2, w3_ref[...], preferred_element_type=jnp.float32)
      + b3_ref[...], 0.0)
  o = jnp.dot(h, w4_ref[...], preferred_element_type=jnp.float32) + b4_ref[...]
  rows = lax.broadcasted_iota(jnp.int32, (NPAD, 8), 0)
  o = jnp.where(rows < N, o, -jnp.inf)
  m = jnp.max(o)
  p = jnp.exp(o - m)
  o_ref[...] = p / jnp.sum(p)


_tc5 = pl.pallas_call(
    _tc5_body,
    out_shape=jax.ShapeDtypeStruct((NPAD, 8), jnp.float32),
)


def kernel(x, edge_index, num_nodes, W1, b1, W2, b2, W3, b3, W4, b4):
  src = edge_index[0]
  dst = edge_index[1]
  pad = jnp.full((EPAD - E,), N, jnp.int32)
  src_p = jnp.concatenate([src, pad]).reshape(NW, NCHUNK, CH)
  dst_p = jnp.concatenate([dst, pad]).reshape(NW, NCHUNK, CH)
  x_p = jnp.pad(x, ((0, NPAD - N), (0, 0)))
  ones_c = jnp.ones((CH, W128), jnp.float32)
  zeros_c = jnp.zeros((ZR, W128), jnp.float32)
  zeros_s = jnp.zeros((ZRS, W128), jnp.float32)

  src_g = src_p.reshape(NCHT, CH)
  dst_g = dst_p.reshape(NCHT, CH)

  degp = _deg_kernel(dst_p, ones_c, zeros_c)
  d0, d1 = degp[0], degp[1]
  gx = _tc1(x_p, d0, d1)
  acc1 = _spmm(gx, src_g, dst_g, zeros_s)
  b2p = jnp.concatenate([b2, jnp.zeros((W128 - 64,), jnp.float32)])
  w3p = jnp.pad(W3, ((0, W128 - 64), (0, 0)))
  g2 = _tc3(acc1[0], acc1[1], gx, d0, d1, W1, b1.reshape(1, -1), W2)
  acc2 = _spmm(g2, src_g, dst_g, zeros_s)
  probs = _tc5(acc2[0], acc2[1], g2, d0, d1, b2p.reshape(1, -1), w3p,
               b3.reshape(1, -1), W4, b4.reshape(1, -1))
  return probs[:N].reshape(1, N * 8)
